# Initial kernel scaffold; baseline (speedup 1.0000x reference)
import jax
import jax.numpy as jnp
from jax.experimental import pallas as pl


def kernel(x, W_enc, b_enc, W_dec, b_dec):
    raise NotImplementedError("write your pallas kernel here")


# speedup vs baseline: 1.0504x; 1.0504x over previous
import functools

import jax
import jax.numpy as jnp
from jax.experimental import pallas as pl
from jax.experimental.pallas import tpu as pltpu

TOKENS = 32
INPUT_DIM = 1024
DICT_SIZE = 32768
BLOCK_D = 2048


def _fused_body(x_ref, we_ref, be_ref, wd_ref, bd_ref, xhat_ref, z_ref):
    i = pl.program_id(0)
    pre = jax.lax.dot_general(
        x_ref[...], we_ref[...],
        dimension_numbers=(((1,), (1,)), ((), ())),
        preferred_element_type=jnp.float32,
    )
    z = jnp.maximum(pre + be_ref[...], 0.0)
    z_ref[...] = z
    part = jax.lax.dot_general(
        z, wd_ref[...],
        dimension_numbers=(((1,), (1,)), ((), ())),
        preferred_element_type=jnp.float32,
    )

    @pl.when(i == 0)
    def _init():
        xhat_ref[...] = part + bd_ref[...]

    @pl.when(i > 0)
    def _acc():
        xhat_ref[...] += part


@functools.partial(jax.jit, static_argnames=())
def kernel(x, W_enc, b_enc, W_dec, b_dec):
    b_enc2 = b_enc.reshape(1, DICT_SIZE)
    b_dec2 = b_dec.reshape(1, INPUT_DIM)
    grid = (DICT_SIZE // BLOCK_D,)
    x_hat, z = pl.pallas_call(
        _fused_body,
        grid=grid,
        in_specs=[
            pl.BlockSpec((TOKENS, INPUT_DIM), lambda i: (0, 0)),
            pl.BlockSpec((BLOCK_D, INPUT_DIM), lambda i: (i, 0)),
            pl.BlockSpec((1, BLOCK_D), lambda i: (0, i)),
            pl.BlockSpec((INPUT_DIM, BLOCK_D), lambda i: (0, i)),
            pl.BlockSpec((1, INPUT_DIM), lambda i: (0, 0)),
        ],
        out_specs=[
            pl.BlockSpec((TOKENS, INPUT_DIM), lambda i: (0, 0)),
            pl.BlockSpec((TOKENS, BLOCK_D), lambda i: (0, i)),
        ],
        out_shape=[
            jax.ShapeDtypeStruct((TOKENS, INPUT_DIM), jnp.float32),
            jax.ShapeDtypeStruct((TOKENS, DICT_SIZE), jnp.float32),
        ],
        compiler_params=pltpu.CompilerParams(
            dimension_semantics=("arbitrary",),
        ),
    )(x, W_enc, b_enc2, W_dec, b_dec2)
    return (x_hat, z)
